# Initial kernel scaffold; baseline (speedup 1.0000x reference)
import jax
import jax.numpy as jnp
from jax.experimental import pallas as pl


def kernel(x, edge_index, eps1, W1, b1, W2, b2, eps2, W3, b3, W4, b4):
    raise NotImplementedError("write your pallas kernel here")


# speedup vs baseline: 20.3045x; 20.3045x over previous
import functools

import jax
import jax.numpy as jnp
from jax import lax
from jax.experimental import pallas as pl
from jax.experimental.pallas import tpu as pltpu
from jax.experimental.pallas import tpu_sc as plsc

NC = 2
NS = 16
NW = NC * NS
L = 16
CHUNK = 128
NBUF = 4


@functools.lru_cache(maxsize=None)
def _make_sc_agg(n, n_pad, k):
    del n
    zr = n_pad // NS
    mesh = plsc.VectorSubcoreMesh(
        core_axis_name="c", subcore_axis_name="s",
        num_cores=NC, num_subcores=NS)

    @functools.partial(
        pl.kernel,
        out_type=jax.ShapeDtypeStruct((NC, n_pad, L), jnp.float32),
        mesh=mesh,
        compiler_params=pltpu.CompilerParams(use_tc_tiling_on_sc=False),
        scratch_types=[
            pltpu.VMEM((k, CHUNK), jnp.int32),
            pltpu.VMEM((k, CHUNK), jnp.int32),
            pltpu.VMEM((NBUF, CHUNK, L), jnp.float32),
            pltpu.VMEM((zr, L), jnp.float32),
            pltpu.VMEM_SHARED((n_pad, L), jnp.float32),
            pltpu.SemaphoreType.DMA((NBUF,)),
        ],
    )
    def agg(y_hbm, src_hbm, dst_hbm, zeros_hbm, out_hbm,
            src_v, dst_v, rows_v, stage_v, acc_sh, gsem):
        c = lax.axis_index("c")
        s = lax.axis_index("s")
        w = c * NS + s

        pltpu.sync_copy(zeros_hbm.at[pl.ds(s * zr, zr)], stage_v)
        pltpu.sync_copy(stage_v, acc_sh.at[pl.ds(s * zr, zr)])
        pltpu.sync_copy(src_hbm.at[w], src_v)
        pltpu.sync_copy(dst_hbm.at[w], dst_v)
        plsc.subcore_barrier()

        for b in range(NBUF):
            pltpu.async_copy(y_hbm.at[src_v.at[b]], rows_v.at[b], gsem.at[b])

        @pl.loop(0, k - NBUF, step=NBUF)
        def _(j):
            for b in range(NBUF):
                pltpu.make_async_copy(
                    y_hbm.at[src_v.at[0]], rows_v.at[b], gsem.at[b]).wait()
                pltpu.sync_copy(rows_v.at[b], acc_sh.at[dst_v.at[j + b]],
                                add=True)
                pltpu.async_copy(y_hbm.at[src_v.at[j + NBUF + b]],
                                 rows_v.at[b], gsem.at[b])

        for b in range(NBUF):
            pltpu.make_async_copy(
                y_hbm.at[src_v.at[0]], rows_v.at[b], gsem.at[b]).wait()
            pltpu.sync_copy(rows_v.at[b], acc_sh.at[dst_v.at[(k - NBUF) + b]],
                            add=True)

        plsc.subcore_barrier()
        pltpu.sync_copy(acc_sh.at[pl.ds(s * zr, zr)], stage_v)
        pltpu.sync_copy(stage_v, out_hbm.at[c].at[pl.ds(s * zr, zr)])

    return agg


def _proj_body(x_ref, w_ref, y_ref):
    y_ref[...] = jnp.dot(x_ref[...], w_ref[...],
                         preferred_element_type=jnp.float32)


def _mid_body(eps_ref, y_ref, pp_ref, b1_ref, w2_ref, b2_ref, w3_ref, z_ref):
    agg = pp_ref[0] + pp_ref[1]
    t = jax.nn.relu((1.0 + eps_ref[...]) * y_ref[...] + agg + b1_ref[...])
    h = jnp.dot(t, w2_ref[...], preferred_element_type=jnp.float32)
    x1 = jax.nn.relu(h + b2_ref[...])
    z_ref[...] = jnp.dot(x1, w3_ref[...], preferred_element_type=jnp.float32)


def _out_body(eps_ref, z_ref, pp_ref, b3_ref, w4_ref, b4_ref, o_ref):
    agg = pp_ref[0] + pp_ref[1]
    t = jax.nn.relu((1.0 + eps_ref[...]) * z_ref[...] + agg + b3_ref[...])
    h = jnp.dot(t, w4_ref[...], preferred_element_type=jnp.float32)
    h = h + b4_ref[...]
    m = jnp.max(h, axis=1, keepdims=True)
    ex = jnp.exp(h - m)
    lse = jnp.log(jnp.sum(ex, axis=1, keepdims=True))
    o_ref[...] = h - m - lse


def kernel(x, edge_index, eps1, W1, b1, W2, b2, eps2, W3, b3, W4, b4):
    n, f_in = x.shape
    e = edge_index.shape[1]
    h = W1.shape[1]
    cdim = W3.shape[1]
    assert h == L and cdim == L

    per_tile = -(-e // (NW * CHUNK * NBUF)) * CHUNK * NBUF
    k = per_tile // CHUNK
    e_pad = NW * per_tile
    pad = e_pad - e
    src = edge_index[0]
    dst = edge_index[1]
    n_pad = -(-(n + 16) // (NS * 8)) * (NS * 8)
    pad_ar = jnp.arange(pad, dtype=jnp.int32)
    src_p = jnp.concatenate([src, pad_ar % n]).reshape(NW, k, CHUNK)
    dst_p = jnp.concatenate([dst, n + pad_ar % (n_pad - n)]).reshape(NW, k, CHUNK)
    zeros = jnp.zeros((n_pad, L), jnp.float32)
    sc_agg = _make_sc_agg(n, n_pad, k)

    blk = 1000
    grid = (n // blk,)
    row_spec = pl.BlockSpec((blk, L), lambda i: (i, 0))
    pp_spec = pl.BlockSpec((NC, blk, L), lambda i: (0, i, 0))
    full = lambda shape: pl.BlockSpec(shape, lambda i: tuple(0 for _ in shape))

    eps1b = jnp.broadcast_to(eps1, (1, L))
    eps2b = jnp.broadcast_to(eps2, (1, L))
    b1b = b1.reshape(1, L)
    b2b = b2.reshape(1, L)
    b3b = b3.reshape(1, L)
    b4b = b4.reshape(1, L)

    y = pl.pallas_call(
        _proj_body,
        grid=grid,
        in_specs=[pl.BlockSpec((blk, f_in), lambda i: (i, 0)),
                  full((f_in, L))],
        out_specs=row_spec,
        out_shape=jax.ShapeDtypeStruct((n, L), jnp.float32),
    )(x, W1)

    pp1 = sc_agg(y, src_p, dst_p, zeros)

    z = pl.pallas_call(
        _mid_body,
        grid=grid,
        in_specs=[full((1, L)), row_spec, pp_spec, full((1, L)),
                  full((L, L)), full((1, L)), full((L, L))],
        out_specs=row_spec,
        out_shape=jax.ShapeDtypeStruct((n, L), jnp.float32),
    )(eps1b, y, pp1, b1b, W2, b2b, W3)

    pp2 = sc_agg(z, src_p, dst_p, zeros)

    out = pl.pallas_call(
        _out_body,
        grid=grid,
        in_specs=[full((1, L)), row_spec, pp_spec, full((1, L)),
                  full((L, L)), full((1, L))],
        out_specs=row_spec,
        out_shape=jax.ShapeDtypeStruct((n, L), jnp.float32),
    )(eps2b, z, pp2, b3b, W4, b4b)

    return out


# speedup vs baseline: 21.0032x; 1.0344x over previous
import functools

import jax
import jax.numpy as jnp
from jax import lax
from jax.experimental import pallas as pl
from jax.experimental.pallas import tpu as pltpu
from jax.experimental.pallas import tpu_sc as plsc

NC = 2
NS = 16
NW = NC * NS
L = 16
CHUNK = 80
NBUF = 5


@functools.lru_cache(maxsize=None)
def _make_sc_agg(n_pad, k):
    zr = n_pad // NS
    mesh = plsc.VectorSubcoreMesh(
        core_axis_name="c", subcore_axis_name="s",
        num_cores=NC, num_subcores=NS)

    @functools.partial(
        pl.kernel,
        out_type=jax.ShapeDtypeStruct((NC, n_pad, L), jnp.float32),
        mesh=mesh,
        compiler_params=pltpu.CompilerParams(use_tc_tiling_on_sc=False),
        scratch_types=[
            pltpu.VMEM((k, CHUNK), jnp.int32),
            pltpu.VMEM((k, CHUNK), jnp.int32),
            pltpu.VMEM((NBUF, CHUNK, L), jnp.float32),
            pltpu.VMEM((zr, L), jnp.float32),
            pltpu.VMEM_SHARED((n_pad, L), jnp.float32),
            pltpu.VMEM_SHARED((n_pad, L), jnp.float32),
            pltpu.SemaphoreType.DMA((NBUF,)),
        ],
    )
    def agg(y_hbm, ei_hbm, zeros_hbm, out_hbm,
            src_v, dst_v, rows_v, stage_v, y_sh, acc_sh, gsem):
        c = lax.axis_index("c")
        s = lax.axis_index("s")
        w = c * NS + s

        pltpu.sync_copy(y_hbm.at[pl.ds(s * zr, zr)], stage_v)
        pltpu.sync_copy(stage_v, y_sh.at[pl.ds(s * zr, zr)])
        pltpu.sync_copy(zeros_hbm.at[pl.ds(s * zr, zr)], stage_v)
        pltpu.sync_copy(stage_v, acc_sh.at[pl.ds(s * zr, zr)])
        pltpu.sync_copy(ei_hbm.at[0].at[w], src_v)
        pltpu.sync_copy(ei_hbm.at[1].at[w], dst_v)
        plsc.subcore_barrier()

        for b in range(NBUF):
            pltpu.async_copy(y_sh.at[src_v.at[b]], rows_v.at[b], gsem.at[b])

        @pl.loop(0, k - NBUF, step=NBUF)
        def _(j):
            for b in range(NBUF):
                pltpu.make_async_copy(
                    y_sh.at[src_v.at[0]], rows_v.at[b], gsem.at[b]).wait()
                pltpu.sync_copy(rows_v.at[b], acc_sh.at[dst_v.at[j + b]],
                                add=True)
                pltpu.async_copy(y_sh.at[src_v.at[j + NBUF + b]],
                                 rows_v.at[b], gsem.at[b])

        for b in range(NBUF):
            pltpu.make_async_copy(
                y_sh.at[src_v.at[0]], rows_v.at[b], gsem.at[b]).wait()
            pltpu.sync_copy(rows_v.at[b], acc_sh.at[dst_v.at[(k - NBUF) + b]],
                            add=True)

        plsc.subcore_barrier()
        pltpu.sync_copy(acc_sh.at[pl.ds(s * zr, zr)], stage_v)
        pltpu.sync_copy(stage_v, out_hbm.at[c].at[pl.ds(s * zr, zr)])

    return agg


def _proj_body(x_ref, w_ref, y_ref):
    y_ref[...] = jnp.dot(x_ref[...], w_ref[...],
                         preferred_element_type=jnp.float32)


def _mid_body(eps_ref, y_ref, pp_ref, b1_ref, w2_ref, b2_ref, w3_ref, z_ref):
    agg = pp_ref[0] + pp_ref[1]
    t = jax.nn.relu((1.0 + eps_ref[...]) * y_ref[...] + agg + b1_ref[...])
    h = jnp.dot(t, w2_ref[...], preferred_element_type=jnp.float32)
    x1 = jax.nn.relu(h + b2_ref[...])
    z_ref[...] = jnp.dot(x1, w3_ref[...], preferred_element_type=jnp.float32)


def _out_body(eps_ref, z_ref, pp_ref, b3_ref, w4_ref, b4_ref, o_ref):
    agg = pp_ref[0] + pp_ref[1]
    t = jax.nn.relu((1.0 + eps_ref[...]) * z_ref[...] + agg + b3_ref[...])
    h = jnp.dot(t, w4_ref[...], preferred_element_type=jnp.float32)
    h = h + b4_ref[...]
    m = jnp.max(h, axis=1, keepdims=True)
    ex = jnp.exp(h - m)
    lse = jnp.log(jnp.sum(ex, axis=1, keepdims=True))
    o_ref[...] = h - m - lse


def kernel(x, edge_index, eps1, W1, b1, W2, b2, eps2, W3, b3, W4, b4):
    n, f_in = x.shape
    e = edge_index.shape[1]
    h = W1.shape[1]
    cdim = W3.shape[1]
    assert h == L and cdim == L

    k = e // (NW * CHUNK)
    assert e == NW * k * CHUNK
    ei = edge_index.reshape(2, NW, k, CHUNK)
    n_pad = -(-n // (NS * 8)) * (NS * 8)
    zeros = jnp.zeros((n_pad, L), jnp.float32)
    sc_agg = _make_sc_agg(n_pad, k)

    pblk = n_pad // NS
    pgrid = (NS,)
    prow_spec = pl.BlockSpec((pblk, L), lambda i: (i, 0))
    ppp_spec = pl.BlockSpec((NC, pblk, L), lambda i: (0, i, 0))
    full = lambda shape: pl.BlockSpec(shape, lambda i: tuple(0 for _ in shape))

    eps1b = jnp.broadcast_to(eps1, (1, L))
    eps2b = jnp.broadcast_to(eps2, (1, L))
    b1b = b1.reshape(1, L)
    b2b = b2.reshape(1, L)
    b3b = b3.reshape(1, L)
    b4b = b4.reshape(1, L)

    y = pl.pallas_call(
        _proj_body,
        grid=pgrid,
        in_specs=[pl.BlockSpec((pblk, f_in), lambda i: (i, 0)),
                  full((f_in, L))],
        out_specs=prow_spec,
        out_shape=jax.ShapeDtypeStruct((n_pad, L), jnp.float32),
    )(x, W1)

    pp1 = sc_agg(y, ei, zeros)

    z = pl.pallas_call(
        _mid_body,
        grid=pgrid,
        in_specs=[full((1, L)), prow_spec, ppp_spec, full((1, L)),
                  full((L, L)), full((1, L)), full((L, L))],
        out_specs=prow_spec,
        out_shape=jax.ShapeDtypeStruct((n_pad, L), jnp.float32),
    )(eps1b, y, pp1, b1b, W2, b2b, W3)

    pp2 = sc_agg(z, ei, zeros)

    blk = 1000
    out = pl.pallas_call(
        _out_body,
        grid=(n // blk,),
        in_specs=[full((1, L)), pl.BlockSpec((blk, L), lambda i: (i, 0)),
                  pl.BlockSpec((NC, blk, L), lambda i: (0, i, 0)),
                  full((1, L)), full((L, L)), full((1, L))],
        out_specs=pl.BlockSpec((blk, L), lambda i: (i, 0)),
        out_shape=jax.ShapeDtypeStruct((n, L), jnp.float32),
    )(eps2b, z, pp2, b3b, W4, b4b)

    return out


# speedup vs baseline: 30.5594x; 1.4550x over previous
import functools

import jax
import jax.numpy as jnp
from jax import lax
from jax.experimental import pallas as pl
from jax.experimental.pallas import tpu as pltpu
from jax.experimental.pallas import tpu_sc as plsc

NC = 2
NS = 16
NW = NC * NS
L = 16
CHUNK = 80
NBUF = 5
G = 8
W128 = G * L


@functools.lru_cache(maxsize=None)
def _make_sc_agg(n_pad, k):
    zr = n_pad // NS
    mesh = plsc.VectorSubcoreMesh(
        core_axis_name="c", subcore_axis_name="s",
        num_cores=NC, num_subcores=NS)

    @functools.partial(
        pl.kernel,
        out_type=jax.ShapeDtypeStruct((NC, n_pad, L), jnp.float32),
        mesh=mesh,
        compiler_params=pltpu.CompilerParams(use_tc_tiling_on_sc=False),
        scratch_types=[
            pltpu.VMEM((k, CHUNK), jnp.int32),
            pltpu.VMEM((k, CHUNK), jnp.int32),
            pltpu.VMEM((NBUF, CHUNK, L), jnp.float32),
            pltpu.VMEM((zr, L), jnp.float32),
            pltpu.VMEM_SHARED((n_pad, L), jnp.float32),
            pltpu.VMEM_SHARED((n_pad, L), jnp.float32),
            pltpu.SemaphoreType.DMA((NBUF,)),
            pltpu.SemaphoreType.DMA((NBUF,)),
        ],
    )
    def agg(y_hbm, ei_hbm, zeros_hbm, out_hbm,
            src_v, dst_v, rows_v, stage_v, y_sh, acc_sh, gsem, ssem):
        c = lax.axis_index("c")
        s = lax.axis_index("s")
        w = c * NS + s

        pltpu.sync_copy(y_hbm.at[pl.ds(s * zr, zr)], stage_v)
        pltpu.sync_copy(stage_v, y_sh.at[pl.ds(s * zr, zr)])
        pltpu.sync_copy(zeros_hbm.at[pl.ds(s * zr, zr)], stage_v)
        pltpu.sync_copy(stage_v, acc_sh.at[pl.ds(s * zr, zr)])
        pltpu.sync_copy(ei_hbm.at[0].at[w], src_v)
        pltpu.sync_copy(ei_hbm.at[1].at[w], dst_v)
        plsc.subcore_barrier()

        def wait_gather(b):
            pltpu.make_async_copy(
                y_sh.at[src_v.at[0]], rows_v.at[b], gsem.at[b]).wait()

        def wait_scatter(b):
            pltpu.make_async_copy(
                rows_v.at[b], acc_sh.at[dst_v.at[0]], ssem.at[b]).wait()

        for b in range(NBUF):
            pltpu.async_copy(y_sh.at[src_v.at[b]], rows_v.at[b], gsem.at[b])

        @pl.loop(0, k - NBUF, step=NBUF)
        def _(j):
            for b in range(NBUF):
                wait_gather(b)
                pltpu.async_copy(rows_v.at[b], acc_sh.at[dst_v.at[j + b]],
                                 ssem.at[b], add=True)
            for b in range(NBUF):
                wait_scatter(b)
                pltpu.async_copy(y_sh.at[src_v.at[j + NBUF + b]],
                                 rows_v.at[b], gsem.at[b])

        for b in range(NBUF):
            wait_gather(b)
            pltpu.async_copy(rows_v.at[b], acc_sh.at[dst_v.at[(k - NBUF) + b]],
                             ssem.at[b], add=True)
        for b in range(NBUF):
            wait_scatter(b)

        plsc.subcore_barrier()
        pltpu.sync_copy(acc_sh.at[pl.ds(s * zr, zr)], stage_v)
        pltpu.sync_copy(stage_v, out_hbm.at[c].at[pl.ds(s * zr, zr)])

    return agg


def _kron(w):
    a, b = w.shape
    t = jnp.tile(w, (G, G))
    ii = lax.broadcasted_iota(jnp.int32, (G * a, G * b), 0) // a
    jj = lax.broadcasted_iota(jnp.int32, (G * a, G * b), 1) // b
    return jnp.where(ii == jj, t, 0.0)


def _group_max(h):
    pos = lax.broadcasted_iota(jnp.int32, h.shape, 1) % L
    neg = jnp.full(h.shape, -1e30, h.dtype)
    m = h
    for s in (1, 2, 4, 8):
        ml = jnp.where(pos <= L - 1 - s, pltpu.roll(m, W128 - s, 1), neg)
        mr = jnp.where(pos >= s, pltpu.roll(m, s, 1), neg)
        m = jnp.maximum(m, jnp.maximum(ml, mr))
    return m


def _proj_body(xg_ref, w_ref, y_ref):
    y_ref[...] = jnp.dot(xg_ref[...], _kron(w_ref[...]),
                         preferred_element_type=jnp.float32)


def _mid_body(eps_ref, y_ref, pp_ref, b1_ref, w2_ref, b2_ref, w3_ref, z_ref):
    agg = pp_ref[0] + pp_ref[1]
    t = jax.nn.relu((1.0 + eps_ref[...]) * y_ref[...] + agg + b1_ref[...])
    h = jnp.dot(t, _kron(w2_ref[...]), preferred_element_type=jnp.float32)
    x1 = jax.nn.relu(h + b2_ref[...])
    z_ref[...] = jnp.dot(x1, _kron(w3_ref[...]),
                         preferred_element_type=jnp.float32)


def _out_body(eps_ref, z_ref, pp_ref, b3_ref, w4_ref, b4_ref, o_ref):
    agg = pp_ref[0] + pp_ref[1]
    t = jax.nn.relu((1.0 + eps_ref[...]) * z_ref[...] + agg + b3_ref[...])
    h = jnp.dot(t, _kron(w4_ref[...]), preferred_element_type=jnp.float32)
    h = h + b4_ref[...]
    m = _group_max(h)
    ex = jnp.exp(h - m)
    ones = jnp.ones((L, L), h.dtype)
    gsum = jnp.dot(ex, _kron(ones), preferred_element_type=jnp.float32)
    o_ref[...] = h - m - jnp.log(gsum)


def kernel(x, edge_index, eps1, W1, b1, W2, b2, eps2, W3, b3, W4, b4):
    n, f_in = x.shape
    e = edge_index.shape[1]
    h = W1.shape[1]
    cdim = W3.shape[1]
    assert h == L and cdim == L

    k = e // (NW * CHUNK)
    assert e == NW * k * CHUNK
    ei = edge_index.reshape(2, NW, k, CHUNK)
    n_pad = -(-n // (NS * 8)) * (NS * 8)
    n_g = n_pad // G
    zeros = jnp.zeros((n_pad, L), jnp.float32)
    sc_agg = _make_sc_agg(n_pad, k)

    gblk = n_g // 2
    grow_spec = pl.BlockSpec((gblk, W128), lambda i: (i, 0))
    gpp_spec = pl.BlockSpec((NC, gblk, W128), lambda i: (0, i, 0))
    full = lambda shape: pl.BlockSpec(shape, lambda i: tuple(0 for _ in shape))

    eps1g = jnp.broadcast_to(eps1, (1, W128))
    eps2g = jnp.broadcast_to(eps2, (1, W128))
    b1g = jnp.tile(b1, G).reshape(1, W128)
    b2g = jnp.tile(b2, G).reshape(1, W128)
    b3g = jnp.tile(b3, G).reshape(1, W128)
    b4g = jnp.tile(b4, G).reshape(1, W128)

    xg = x.reshape(n // G, G * f_in)
    y_g = pl.pallas_call(
        _proj_body,
        grid=(2,),
        in_specs=[pl.BlockSpec((gblk, G * f_in), lambda i: (i, 0)),
                  full((f_in, L))],
        out_specs=grow_spec,
        out_shape=jax.ShapeDtypeStruct((n_g, W128), jnp.float32),
    )(xg, W1)

    pp1 = sc_agg(y_g.reshape(n_pad, L), ei, zeros)

    z_g = pl.pallas_call(
        _mid_body,
        grid=(2,),
        in_specs=[full((1, W128)), grow_spec, gpp_spec, full((1, W128)),
                  full((L, L)), full((1, W128)), full((L, L))],
        out_specs=grow_spec,
        out_shape=jax.ShapeDtypeStruct((n_g, W128), jnp.float32),
    )(eps1g, y_g, pp1.reshape(NC, n_g, W128), b1g, W2, b2g, W3)

    pp2 = sc_agg(z_g.reshape(n_pad, L), ei, zeros)

    o_g = pl.pallas_call(
        _out_body,
        grid=(2,),
        in_specs=[full((1, W128)), grow_spec, gpp_spec,
                  full((1, W128)), full((L, L)), full((1, W128))],
        out_specs=grow_spec,
        out_shape=jax.ShapeDtypeStruct((n_g, W128), jnp.float32),
    )(eps2g, z_g, pp2.reshape(NC, n_g, W128), b3g, W4, b4g)

    return o_g.reshape(n_pad, L)[:n]


# speedup vs baseline: 31.3432x; 1.0256x over previous
import functools

import jax
import jax.numpy as jnp
from jax import lax
from jax.experimental import pallas as pl
from jax.experimental.pallas import tpu as pltpu
from jax.experimental.pallas import tpu_sc as plsc

NC = 2
NS = 16
NW = NC * NS
L = 16
CHUNK = 128
NBUF = 10
G = 8
W128 = G * L


@functools.lru_cache(maxsize=None)
def _make_sc_agg(n_pad, k):
    zr = n_pad // NS
    mesh = plsc.VectorSubcoreMesh(
        core_axis_name="c", subcore_axis_name="s",
        num_cores=NC, num_subcores=NS)

    @functools.partial(
        pl.kernel,
        out_type=jax.ShapeDtypeStruct((NC, n_pad, L), jnp.float32),
        mesh=mesh,
        compiler_params=pltpu.CompilerParams(use_tc_tiling_on_sc=False),
        scratch_types=[
            pltpu.VMEM((k, CHUNK), jnp.int32),
            pltpu.VMEM((k, CHUNK), jnp.int32),
            pltpu.VMEM((NBUF, CHUNK, L), jnp.float32),
            pltpu.VMEM((zr, L), jnp.float32),
            pltpu.VMEM_SHARED((n_pad, L), jnp.float32),
            pltpu.SemaphoreType.DMA((NBUF,)),
            pltpu.SemaphoreType.DMA((NBUF,)),
        ],
    )
    def agg(y_hbm, ei_hbm, zeros_hbm, out_hbm,
            src_v, dst_v, rows_v, stage_v, acc_sh, gsem, ssem):
        c = lax.axis_index("c")
        s = lax.axis_index("s")
        w = c * NS + s

        pltpu.sync_copy(zeros_hbm.at[pl.ds(s * zr, zr)], stage_v)
        pltpu.sync_copy(stage_v, acc_sh.at[pl.ds(s * zr, zr)])
        pltpu.sync_copy(ei_hbm.at[0].at[w], src_v)
        pltpu.sync_copy(ei_hbm.at[1].at[w], dst_v)
        plsc.subcore_barrier()

        def wait_gather(b):
            pltpu.make_async_copy(
                y_hbm.at[src_v.at[0]], rows_v.at[b], gsem.at[b]).wait()

        def wait_scatter(b):
            pltpu.make_async_copy(
                rows_v.at[b], acc_sh.at[dst_v.at[0]], ssem.at[b]).wait()

        for b in range(NBUF):
            pltpu.async_copy(y_hbm.at[src_v.at[b]], rows_v.at[b], gsem.at[b])

        @pl.loop(0, k - NBUF, step=NBUF)
        def _(j):
            for b in range(NBUF):
                wait_gather(b)
                pltpu.async_copy(rows_v.at[b], acc_sh.at[dst_v.at[j + b]],
                                 ssem.at[b], add=True)
            for b in range(NBUF):
                wait_scatter(b)
                pltpu.async_copy(y_hbm.at[src_v.at[j + NBUF + b]],
                                 rows_v.at[b], gsem.at[b])

        for b in range(NBUF):
            wait_gather(b)
            pltpu.async_copy(rows_v.at[b], acc_sh.at[dst_v.at[(k - NBUF) + b]],
                             ssem.at[b], add=True)
        for b in range(NBUF):
            wait_scatter(b)

        plsc.subcore_barrier()
        pltpu.sync_copy(acc_sh.at[pl.ds(s * zr, zr)], stage_v)
        pltpu.sync_copy(stage_v, out_hbm.at[c].at[pl.ds(s * zr, zr)])

    return agg


def _kron(w):
    a, b = w.shape
    t = jnp.tile(w, (G, G))
    ii = lax.broadcasted_iota(jnp.int32, (G * a, G * b), 0) // a
    jj = lax.broadcasted_iota(jnp.int32, (G * a, G * b), 1) // b
    return jnp.where(ii == jj, t, 0.0)


def _group_max(h):
    pos = lax.broadcasted_iota(jnp.int32, h.shape, 1) % L
    neg = jnp.full(h.shape, -1e30, h.dtype)
    m = h
    for s in (1, 2, 4, 8):
        ml = jnp.where(pos <= L - 1 - s, pltpu.roll(m, W128 - s, 1), neg)
        mr = jnp.where(pos >= s, pltpu.roll(m, s, 1), neg)
        m = jnp.maximum(m, jnp.maximum(ml, mr))
    return m


def _proj_body(xg_ref, w_ref, y_ref):
    y_ref[...] = jnp.dot(xg_ref[...], _kron(w_ref[...]),
                         preferred_element_type=jnp.float32)


def _mid_body(eps_ref, y_ref, pp_ref, b1_ref, w2_ref, b2_ref, w3_ref, z_ref):
    agg = pp_ref[0] + pp_ref[1]
    t = jax.nn.relu((1.0 + eps_ref[...]) * y_ref[...] + agg + b1_ref[...])
    h = jnp.dot(t, _kron(w2_ref[...]), preferred_element_type=jnp.float32)
    x1 = jax.nn.relu(h + b2_ref[...])
    z_ref[...] = jnp.dot(x1, _kron(w3_ref[...]),
                         preferred_element_type=jnp.float32)


def _out_body(eps_ref, z_ref, pp_ref, b3_ref, w4_ref, b4_ref, o_ref):
    agg = pp_ref[0] + pp_ref[1]
    t = jax.nn.relu((1.0 + eps_ref[...]) * z_ref[...] + agg + b3_ref[...])
    h = jnp.dot(t, _kron(w4_ref[...]), preferred_element_type=jnp.float32)
    h = h + b4_ref[...]
    m = _group_max(h)
    ex = jnp.exp(h - m)
    ones = jnp.ones((L, L), h.dtype)
    gsum = jnp.dot(ex, _kron(ones), preferred_element_type=jnp.float32)
    o_ref[...] = h - m - jnp.log(gsum)


def kernel(x, edge_index, eps1, W1, b1, W2, b2, eps2, W3, b3, W4, b4):
    n, f_in = x.shape
    e = edge_index.shape[1]
    h = W1.shape[1]
    cdim = W3.shape[1]
    assert h == L and cdim == L

    n_pad = -(-n // (NS * 8)) * (NS * 8)
    n_g = n_pad // G
    k = -(-e // (NW * CHUNK * NBUF)) * NBUF
    e_pad = NW * k * CHUNK
    if e_pad > e:
        pad_ar = jnp.arange(e_pad - e, dtype=jnp.int32)
        pad_blk = jnp.stack([pad_ar % n, n + pad_ar % (n_pad - n)])
        ei = jnp.concatenate([edge_index, pad_blk], axis=1)
    else:
        ei = edge_index
    ei = ei.reshape(2, NW, k, CHUNK)
    zeros = jnp.zeros((n_pad, L), jnp.float32)
    sc_agg = _make_sc_agg(n_pad, k)

    gblk = n_g // 2
    grow_spec = pl.BlockSpec((gblk, W128), lambda i: (i, 0))
    gpp_spec = pl.BlockSpec((NC, gblk, W128), lambda i: (0, i, 0))
    full = lambda shape: pl.BlockSpec(shape, lambda i: tuple(0 for _ in shape))

    eps1g = jnp.broadcast_to(eps1, (1, W128))
    eps2g = jnp.broadcast_to(eps2, (1, W128))
    b1g = jnp.tile(b1, G).reshape(1, W128)
    b2g = jnp.tile(b2, G).reshape(1, W128)
    b3g = jnp.tile(b3, G).reshape(1, W128)
    b4g = jnp.tile(b4, G).reshape(1, W128)

    xg = x.reshape(n // G, G * f_in)
    y_g = pl.pallas_call(
        _proj_body,
        grid=(2,),
        in_specs=[pl.BlockSpec((gblk, G * f_in), lambda i: (i, 0)),
                  full((f_in, L))],
        out_specs=grow_spec,
        out_shape=jax.ShapeDtypeStruct((n_g, W128), jnp.float32),
    )(xg, W1)

    pp1 = sc_agg(y_g.reshape(n_pad, L), ei, zeros)

    z_g = pl.pallas_call(
        _mid_body,
        grid=(2,),
        in_specs=[full((1, W128)), grow_spec, gpp_spec, full((1, W128)),
                  full((L, L)), full((1, W128)), full((L, L))],
        out_specs=grow_spec,
        out_shape=jax.ShapeDtypeStruct((n_g, W128), jnp.float32),
    )(eps1g, y_g, pp1.reshape(NC, n_g, W128), b1g, W2, b2g, W3)

    pp2 = sc_agg(z_g.reshape(n_pad, L), ei, zeros)

    o_g = pl.pallas_call(
        _out_body,
        grid=(2,),
        in_specs=[full((1, W128)), grow_spec, gpp_spec,
                  full((1, W128)), full((L, L)), full((1, W128))],
        out_specs=grow_spec,
        out_shape=jax.ShapeDtypeStruct((n_g, W128), jnp.float32),
    )(eps2g, z_g, pp2.reshape(NC, n_g, W128), b3g, W4, b4g)

    return o_g.reshape(n_pad, L)[:n]
